# baseline (speedup 1.0000x reference)
import functools

import jax
import jax.numpy as jnp
from jax import lax
from jax.experimental import pallas as pl
from jax.experimental.pallas import tpu as pltpu
from jax.experimental.pallas import tpu_sc as plsc

B = 16384
D = 32
IDX_CHUNK = 128

_info = plsc.get_sparse_core_info()
_NC = _info.num_cores
_NS = _info.num_subcores
NW = _NC * _NS
B_PER_W = B // NW
CHUNKS = B_PER_W // IDX_CHUNK

_mesh = plsc.VectorSubcoreMesh(core_axis_name="c", subcore_axis_name="s")


@functools.partial(
    pl.kernel,
    mesh=_mesh,
    out_type=jax.ShapeDtypeStruct((B, D), jnp.float32),
    scratch_types=[
        pltpu.VMEM((CHUNKS, IDX_CHUNK), jnp.int32),
        pltpu.VMEM((B_PER_W, D), jnp.float32),
        pltpu.SemaphoreType.DMA,
    ],
    compiler_params=pltpu.CompilerParams(use_tc_tiling_on_sc=False),
)
def _gather_kernel(idx_hbm, table_hbm, out_hbm, idx_v, rows_v, sem):
    wid = lax.axis_index("s") * _NC + lax.axis_index("c")
    base = wid * B_PER_W
    pltpu.sync_copy(idx_hbm.at[pl.ds(wid * CHUNKS, CHUNKS)], idx_v)
    copies = []
    for j in range(CHUNKS):
        copies.append(
            pltpu.async_copy(
                table_hbm.at[idx_v.at[j]],
                rows_v.at[pl.ds(j * IDX_CHUNK, IDX_CHUNK)],
                sem,
            )
        )
    for c in copies:
        c.wait()
    pltpu.sync_copy(rows_v, out_hbm.at[pl.ds(base, B_PER_W)])


def kernel(actionList, table):
    idx = actionList[:, -1].astype(jnp.int32).reshape(NW * CHUNKS, IDX_CHUNK)
    return _gather_kernel(idx, table)


# speedup vs baseline: 4.0097x; 4.0097x over previous
import functools

import jax
import jax.numpy as jnp
from jax import lax
from jax.experimental import pallas as pl
from jax.experimental.pallas import tpu as pltpu
from jax.experimental.pallas import tpu_sc as plsc

B = 16384
D = 32
LANES = 128

_info = plsc.get_sparse_core_info()
_NC = _info.num_cores
_NS = _info.num_subcores
NW = _NC * _NS
B_PER_W = B // NW
NSLOT = 8
GVEC = 16
GROUPS = B_PER_W // GVEC

_mesh = plsc.VectorSubcoreMesh(core_axis_name="c", subcore_axis_name="s")


@functools.partial(
    pl.kernel,
    mesh=_mesh,
    out_type=jax.ShapeDtypeStruct((B, D), jnp.float32),
    scratch_types=[
        pltpu.VMEM((B_PER_W,), jnp.int32),
        pltpu.VMEM((NSLOT, D, LANES), jnp.float32),
        pltpu.VMEM((B_PER_W, D), jnp.float32),
        pltpu.SemaphoreType.DMA((NSLOT,)),
    ],
    compiler_params=pltpu.CompilerParams(needs_layout_passes=False),
)
def _gather_kernel(idx_hbm, table_t_hbm, out_hbm, idx_v, stage_v, out_v, sems):
    wid = lax.axis_index("s") * _NC + lax.axis_index("c")
    base = wid * B_PER_W
    pltpu.sync_copy(idx_hbm.at[wid], idx_v)

    rows_lo = lax.iota(jnp.int32, 16)
    rows_hi = rows_lo + 16

    def fire(v, slot):
        col0 = pl.multiple_of((v >> 7) * LANES, LANES)
        pltpu.async_copy(
            table_t_hbm.at[:, pl.ds(col0, LANES)],
            stage_v.at[slot],
            sems.at[slot],
        )

    vec0 = idx_v[pl.ds(0, GVEC)]
    for k in range(NSLOT):
        fire(vec0[k], k)

    def body(g, carry):
        vec = idx_v[pl.ds(g * GVEC, GVEC)]
        vec_next = idx_v[pl.ds((g + 1) * GVEC % B_PER_W, GVEC)]
        last = g == GROUPS - 1
        for k in range(GVEC):
            slot = k % NSLOT
            pltpu.make_async_copy(
                table_t_hbm.at[:, pl.ds(0, LANES)],
                stage_v.at[slot],
                sems.at[slot],
            ).wait()
            lane = jnp.full((16,), vec[k] & 127, jnp.int32)
            lo = plsc.load_gather(stage_v.at[slot], [rows_lo, lane])
            hi = plsc.load_gather(stage_v.at[slot], [rows_hi, lane])
            j = g * GVEC + k
            out_v[j, pl.ds(0, 16)] = lo
            out_v[j, pl.ds(16, 16)] = hi
            if k < NSLOT:
                fire(vec[k + NSLOT], slot)
            else:

                @pl.when(jnp.logical_not(last))
                def _():
                    fire(vec_next[k - NSLOT], slot)

        return carry

    lax.fori_loop(0, GROUPS, body, 0)
    pltpu.sync_copy(out_v, out_hbm.at[pl.ds(base, B_PER_W)])


def kernel(actionList, table):
    idx = actionList[:, -1].astype(jnp.int32).reshape(NW, B_PER_W)
    return _gather_kernel(idx, table.T)
